# baseline (speedup 1.0000x reference)
import functools

import jax
import jax.numpy as jnp
from jax import lax
from jax.experimental import pallas as pl
from jax.experimental.pallas import tpu as pltpu
from jax.experimental.pallas import tpu_sc as plsc

_NG, _NJ, _H = 50000, 100000, 128
_E, _EP = 200000, 204800
_C = 4096
_NC, _NS = 2, 16
_SENT = 1 << 30


def _sc_mesh():
    return plsc.VectorSubcoreMesh(core_axis_name="c", subcore_axis_name="s",
                                  num_cores=_NC, num_subcores=_NS)


def _segsum(vals, src_p, dst_p, ndst):
    npass = -(-ndst // (_NC * _C))
    ndst_pad = npass * _NC * _C
    per_tile = _EP // _NS
    nbatch = per_tile // 128
    share = _C // _NS

    def body(vals_hbm, src_hbm, dst_hbm, sum_out, cnt_out,
             src_v, dst_v, sidx_v, rows_v, ones_v, zrow_v,
             acc_sum, acc_cnt, sem):
        c = lax.axis_index("c")
        s = lax.axis_index("s")
        lanes = lax.iota(jnp.int32, 16)
        onepat = jnp.where(lanes == 0, 1.0, 0.0).astype(jnp.float32)
        z16 = jnp.zeros((16,), jnp.float32)

        def init_row(i, carry):
            ones_v[i, pl.ds(0, 16)] = onepat
            for k in range(8):
                zrow_v[i, pl.ds(k * 16, 16)] = z16
                if k > 0:
                    ones_v[i, pl.ds(k * 16, 16)] = z16
            return carry
        lax.fori_loop(0, 128, init_row, 0)

        for p in range(npass):
            qbase = (p * _NC + c) * _C
            for k in range(share // 128):
                pltpu.sync_copy(zrow_v, acc_sum.at[pl.ds(s * share + k * 128, 128)])
                pltpu.sync_copy(zrow_v, acc_cnt.at[pl.ds(s * share + k * 128, 128)])
            plsc.subcore_barrier()

            def batch(b, carry):
                off = s * per_tile + b * 128
                pltpu.sync_copy(src_hbm.at[pl.ds(off, 128)], src_v)
                pltpu.sync_copy(dst_hbm.at[pl.ds(off, 128)], dst_v)
                for g in range(8):
                    d = dst_v[pl.ds(g * 16, 16)]
                    m = (d >= qbase) & (d < qbase + _C)
                    sidx_v[pl.ds(g * 16, 16)] = jnp.where(m, d - qbase, _C)
                pltpu.async_copy(vals_hbm.at[src_v], rows_v, sem).wait()
                pltpu.sync_copy(rows_v, acc_sum.at[sidx_v], add=True)
                pltpu.sync_copy(ones_v, acc_cnt.at[sidx_v], add=True)
                return carry
            lax.fori_loop(0, nbatch, batch, 0)

            plsc.subcore_barrier()
            dstbase = qbase + s * share
            pltpu.sync_copy(acc_sum.at[pl.ds(s * share, share)],
                            sum_out.at[pl.ds(dstbase, share)])
            pltpu.sync_copy(acc_cnt.at[pl.ds(s * share, share)],
                            cnt_out.at[pl.ds(dstbase, share)])
            plsc.subcore_barrier()

    kfn = pl.kernel(
        body,
        out_type=[jax.ShapeDtypeStruct((ndst_pad, _H), jnp.float32),
                  jax.ShapeDtypeStruct((ndst_pad, _H), jnp.float32)],
        mesh=_sc_mesh(),
        scratch_types=[
            pltpu.VMEM((128,), jnp.int32),
            pltpu.VMEM((128,), jnp.int32),
            pltpu.VMEM((128,), jnp.int32),
            pltpu.VMEM((128, _H), jnp.float32),
            pltpu.VMEM((128, _H), jnp.float32),
            pltpu.VMEM((128, _H), jnp.float32),
            pltpu.VMEM_SHARED((_C + 16, _H), jnp.float32),
            pltpu.VMEM_SHARED((_C + 16, _H), jnp.float32),
            pltpu.SemaphoreType.DMA,
        ],
    )
    return kfn(vals, src_p, dst_p)


def _edge_gather(u1, u2, src_p, dst_p):
    per_tile = _EP // (_NC * _NS)
    nbatch = per_tile // 128

    def body(u1_hbm, u2_hbm, src_hbm, dst_hbm, e1_out, e2_out,
             idx_v, rows_v, sem):
        c = lax.axis_index("c")
        s = lax.axis_index("s")
        wid = s * _NC + c

        def batch(b, carry):
            off = wid * per_tile + b * 128
            pltpu.sync_copy(src_hbm.at[pl.ds(off, 128)], idx_v)
            pltpu.async_copy(u1_hbm.at[idx_v], rows_v, sem).wait()
            pltpu.sync_copy(rows_v, e1_out.at[pl.ds(off, 128)])
            pltpu.sync_copy(dst_hbm.at[pl.ds(off, 128)], idx_v)
            pltpu.async_copy(u2_hbm.at[idx_v], rows_v, sem).wait()
            pltpu.sync_copy(rows_v, e2_out.at[pl.ds(off, 128)])
            return carry
        lax.fori_loop(0, nbatch, batch, 0)

    kfn = pl.kernel(
        body,
        out_type=[jax.ShapeDtypeStruct((_EP, _H), jnp.float32),
                  jax.ShapeDtypeStruct((_EP, _H), jnp.float32)],
        mesh=_sc_mesh(),
        scratch_types=[
            pltpu.VMEM((128,), jnp.int32),
            pltpu.VMEM((128, _H), jnp.float32),
            pltpu.SemaphoreType.DMA,
        ],
    )
    return kfn(u1, u2, src_p, dst_p)


def _enc(x8, wx, b):
    n = x8.shape[0]
    B = 1000

    def body(x_ref, w_ref, b_ref, h_ref, c_ref):
        a = jnp.dot(x_ref[...], w_ref[...],
                    preferred_element_type=jnp.float32) + b_ref[...]
        i, f, g, o = (a[:, :_H], a[:, _H:2 * _H],
                      a[:, 2 * _H:3 * _H], a[:, 3 * _H:])
        c1 = jax.nn.sigmoid(i) * jnp.tanh(g)
        h_ref[...] = jax.nn.sigmoid(o) * jnp.tanh(c1)
        c_ref[...] = c1

    return pl.pallas_call(
        body,
        grid=(n // B,),
        in_specs=[pl.BlockSpec((B, 8), lambda i: (i, 0)),
                  pl.BlockSpec((8, 4 * _H), lambda i: (0, 0)),
                  pl.BlockSpec((1, 4 * _H), lambda i: (0, 0))],
        out_specs=[pl.BlockSpec((B, _H), lambda i: (i, 0))] * 2,
        out_shape=[jax.ShapeDtypeStruct((n, _H), jnp.float32)] * 2,
    )(x8, wx, b)


def _dec_joint(x8, h1, c1, m1s, m1c, m2s, m2c,
               wx, wl1, wl2, wr, b, linj8, bj8, ed1a, ed1b, ed1bias):
    B = 1000

    def body(x_ref, h_ref, c_ref, m1s_ref, m1c_ref, m2s_ref, m2c_ref,
             wx_ref, wl1_ref, wl2_ref, wr_ref, b_ref, linj_ref, bj_ref,
             e1a_ref, e1b_ref, e1bias_ref, u1_ref, u2_ref, yj_ref):
        m1 = m1s_ref[...] / jnp.maximum(m1c_ref[:, 0:1], 1.0)
        m2 = m2s_ref[...] / jnp.maximum(m2c_ref[:, 0:1], 1.0)
        dot = functools.partial(jnp.dot, preferred_element_type=jnp.float32)
        a = (dot(x_ref[...], wx_ref[...]) + dot(m1, wl1_ref[...])
             + dot(m2, wl2_ref[...]) + dot(h_ref[...], wr_ref[...])
             + b_ref[...])
        i, f, g, o = (a[:, :_H], a[:, _H:2 * _H],
                      a[:, 2 * _H:3 * _H], a[:, 3 * _H:])
        c2 = jax.nn.sigmoid(f) * c_ref[...] + jax.nn.sigmoid(i) * jnp.tanh(g)
        h2 = jax.nn.sigmoid(o) * jnp.tanh(c2)
        u1_ref[...] = dot(h2, e1a_ref[...]) + e1bias_ref[...]
        u2_ref[...] = dot(h2, e1b_ref[...])
        yj_ref[...] = jax.nn.sigmoid(dot(h2, linj_ref[...]) + bj_ref[...]) - 0.5

    full = lambda shape: pl.BlockSpec(shape, lambda i: (0, 0))
    row = lambda w: pl.BlockSpec((B, w), lambda i: (i, 0))
    return pl.pallas_call(
        body,
        grid=(_NJ // B,),
        in_specs=[row(8), row(_H), row(_H), row(_H), row(_H), row(_H), row(_H),
                  full((8, 4 * _H)), full((_H, 4 * _H)), full((_H, 4 * _H)),
                  full((_H, 4 * _H)), full((1, 4 * _H)), full((_H, 8)),
                  full((1, 8)), full((_H, _H)), full((_H, _H)), full((1, _H))],
        out_specs=[row(_H), row(_H), row(8)],
        out_shape=[jax.ShapeDtypeStruct((_NJ, _H), jnp.float32),
                   jax.ShapeDtypeStruct((_NJ, _H), jnp.float32),
                   jax.ShapeDtypeStruct((_NJ, 8), jnp.float32)],
    )(x8, h1, c1, m1s, m1c, m2s, m2c, wx, wl1, wl2, wr, b, linj8, bj8,
      ed1a, ed1b, ed1bias)


def _dec_grain(x8, h1, c1, ms, mc, wx, wl, wr, b, ling8, bg8):
    B = 1000

    def body(x_ref, h_ref, c_ref, ms_ref, mc_ref,
             wx_ref, wl_ref, wr_ref, b_ref, ling_ref, bg_ref,
             g8_ref, tot_ref):
        m = ms_ref[...] / jnp.maximum(mc_ref[:, 0:1], 1.0)
        dot = functools.partial(jnp.dot, preferred_element_type=jnp.float32)
        a = (dot(x_ref[...], wx_ref[...]) + dot(m, wl_ref[...])
             + dot(h_ref[...], wr_ref[...]) + b_ref[...])
        i, f, g, o = (a[:, :_H], a[:, _H:2 * _H],
                      a[:, 2 * _H:3 * _H], a[:, 3 * _H:])
        c2 = jax.nn.sigmoid(f) * c_ref[...] + jax.nn.sigmoid(i) * jnp.tanh(g)
        h2 = jax.nn.sigmoid(o) * jnp.tanh(c2)
        y8 = dot(h2, ling_ref[...]) + bg_ref[...]
        col = lax.broadcasted_iota(jnp.int32, (B, 8), 1)
        x3 = x_ref[:, 3:4]
        relu_shift = jnp.maximum(y8 + x3, 0.0)
        relu_y = jnp.maximum(y8, 0.0)
        g8_ref[...] = jnp.where(col == 0, relu_shift,
                                jnp.where(col == 1, relu_y, 0.0))
        part = jnp.sum(jnp.where(col == 0, relu_shift, 0.0))

        @pl.when(pl.program_id(0) == 0)
        def _():
            tot_ref[0, 0] = 0.0
        tot_ref[0, 0] += part

    full = lambda shape: pl.BlockSpec(shape, lambda i: (0, 0))
    row = lambda w: pl.BlockSpec((B, w), lambda i: (i, 0))
    return pl.pallas_call(
        body,
        grid=(_NG // B,),
        in_specs=[row(8), row(_H), row(_H), row(_H), row(_H),
                  full((8, 4 * _H)), full((_H, 4 * _H)), full((_H, 4 * _H)),
                  full((1, 4 * _H)), full((_H, 8)), full((1, 8))],
        out_specs=[row(8),
                   pl.BlockSpec((1, 1), lambda i: (0, 0),
                                memory_space=pltpu.SMEM)],
        out_shape=[jax.ShapeDtypeStruct((_NG, 8), jnp.float32),
                   jax.ShapeDtypeStruct((1, 1), jnp.float32)],
    )(x8, h1, c1, ms, mc, wx, wl, wr, b, ling8, bg8)


def _area(g8, x8, tot):
    B = 1000

    def body(g8_ref, x_ref, tot_ref, out_ref):
        t = jnp.maximum(tot_ref[0, 0], 1e-12)
        col = lax.broadcasted_iota(jnp.int32, (B, 8), 1)
        x3 = x_ref[:, 3:4]
        out_ref[...] = jnp.where(col == 0, g8_ref[...] / t - x3, g8_ref[...])

    return pl.pallas_call(
        body,
        grid=(_NG // B,),
        in_specs=[pl.BlockSpec((B, 8), lambda i: (i, 0)),
                  pl.BlockSpec((B, 8), lambda i: (i, 0)),
                  pl.BlockSpec((1, 1), lambda i: (0, 0),
                               memory_space=pltpu.SMEM)],
        out_specs=pl.BlockSpec((B, 8), lambda i: (i, 0)),
        out_shape=jax.ShapeDtypeStruct((_NG, 8), jnp.float32),
    )(g8, x8, tot)


def _edge_final(e1, e2, w2pad, b2pad):
    B = 8192

    def body(e1_ref, e2_ref, w2_ref, b2_ref, z_ref):
        t = jnp.maximum(e1_ref[...] + e2_ref[...], 0.0)
        z_ref[...] = jax.nn.sigmoid(
            jnp.dot(t, w2_ref[...], preferred_element_type=jnp.float32)
            + b2_ref[...])

    return pl.pallas_call(
        body,
        grid=(_EP // B,),
        in_specs=[pl.BlockSpec((B, _H), lambda i: (i, 0)),
                  pl.BlockSpec((B, _H), lambda i: (i, 0)),
                  pl.BlockSpec((_H, 8), lambda i: (0, 0)),
                  pl.BlockSpec((1, 8), lambda i: (0, 0))],
        out_specs=pl.BlockSpec((B, 8), lambda i: (i, 0)),
        out_shape=jax.ShapeDtypeStruct((_EP, 8), jnp.float32),
    )(e1, e2, w2pad, b2pad)


def kernel(x_grain, x_joint, edge_index_gj, edge_index_jg, edge_index_jj,
           params):
    f32 = jnp.float32
    xg8 = x_grain
    xj8 = jnp.pad(x_joint, ((0, 0), (0, 2)))

    def cat_w(cell, kind, pad_rows=0):
        w = jnp.concatenate([params[f'{cell}_{g}_{kind}'] for g in 'ifco'],
                            axis=1)
        if pad_rows:
            w = jnp.pad(w, ((0, pad_rows), (0, 0)))
        return w

    def cat_b(cell, kind):
        return jnp.concatenate(
            [params[f'{cell}_{g}_{kind}'] for g in 'ifco'])[None, :]

    h_g1, c_g1 = _enc(xg8, cat_w('enc', 'Wx_g'), cat_b('enc', 'b_g'))
    h_j1, c_j1 = _enc(xj8, cat_w('enc', 'Wx_j', pad_rows=2),
                      cat_b('enc', 'b_j'))

    npad = _EP - _E
    pad0 = jnp.zeros((npad,), jnp.int32)
    sent = jnp.full((npad,), _SENT, jnp.int32)
    gj_src = jnp.concatenate([edge_index_gj[0], pad0])
    gj_dst = jnp.concatenate([edge_index_gj[1], sent])
    jg_src = jnp.concatenate([edge_index_jg[0], pad0])
    jg_dst = jnp.concatenate([edge_index_jg[1], sent])
    jj_src = jnp.concatenate([edge_index_jj[0], pad0])
    jj_dst = jnp.concatenate([edge_index_jj[1], sent])
    jj_src0 = jnp.concatenate([edge_index_jj[0], pad0])
    jj_dst0 = jnp.concatenate([edge_index_jj[1], pad0])

    sj1, nj1 = _segsum(h_g1, gj_src, gj_dst, 50000)
    sj2, nj2 = _segsum(h_j1, jj_src, jj_dst, _NJ)
    sg, ng = _segsum(h_j1, jg_src, jg_dst, _NG)

    zf = jnp.zeros((_NJ - sj1.shape[0], _H), f32)
    zc = jnp.zeros((_NJ - nj1.shape[0], _H), f32)
    m1s = jnp.concatenate([sj1, zf])
    m1c = jnp.concatenate([nj1, zc])
    m2s, m2c = sj2[:_NJ], nj2[:_NJ]
    mgs, mgc = sg[:_NG], ng[:_NG]

    wx_j = cat_w('dec', 'Wx_j', pad_rows=2)
    wl_gj = cat_w('dec', 'Wl_gj')
    wl_jj = cat_w('dec', 'Wl_jj')
    wr_j = jnp.concatenate(
        [params[f'dec_{g}_Wr_gj'] + params[f'dec_{g}_Wr_jj'] for g in 'ifco'],
        axis=1)
    b_j = cat_b('dec', 'b_j')
    wx_g = cat_w('dec', 'Wx_g')
    wl_jg = cat_w('dec', 'Wl_jg')
    wr_g = cat_w('dec', 'Wr_jg')
    b_g = cat_b('dec', 'b_g')

    linj8 = jnp.pad(params['lin_j_W'], ((0, 0), (0, 5)))
    bj8 = jnp.pad(params['lin_j_b'], (0, 5))[None, :]
    ed1a = params['ed1_W'][:_H]
    ed1b = params['ed1_W'][_H:]
    ed1bias = params['ed1_b'][None, :]
    ling8 = jnp.pad(params['lin_g_W'], ((0, 0), (0, 6)))
    bg8 = jnp.pad(params['lin_g_b'], (0, 6))[None, :]
    w2pad = jnp.pad(params['ed2_W'], ((0, 0), (0, 7)))
    b2pad = jnp.pad(params['ed2_b'], (0, 7))[None, :]

    u1, u2, yj8 = _dec_joint(xj8, h_j1, c_j1, m1s, m1c, m2s, m2c,
                             wx_j, wl_gj, wl_jj, wr_j, b_j,
                             linj8, bj8, ed1a, ed1b, ed1bias)
    g8, tot = _dec_grain(xg8, h_g1, c_g1, mgs, mgc,
                         wx_g, wl_jg, wr_g, b_g, ling8, bg8)
    ya8 = _area(g8, xg8, tot)

    e1, e2 = _edge_gather(u1, u2, jj_src0, jj_dst0)
    z8 = _edge_final(e1, e2, w2pad, b2pad)

    return ya8[:, :2], yj8[:, :3], z8[:_E, 0]


# speedup vs baseline: 1.0243x; 1.0243x over previous
import functools

import jax
import jax.numpy as jnp
from jax import lax
from jax.experimental import pallas as pl
from jax.experimental.pallas import tpu as pltpu
from jax.experimental.pallas import tpu_sc as plsc

_NG, _NJ, _H = 50000, 100000, 128
_E, _EP = 200000, 204800
_C = 4096
_NC, _NS = 2, 16
_SENT = 1 << 30
_CAP = 6144
_SEG = 1280


def _sc_mesh():
    return plsc.VectorSubcoreMesh(core_axis_name="c", subcore_axis_name="s",
                                  num_cores=_NC, num_subcores=_NS)


def _segsum(vals, aux, src_p, dst_p, ndst):
    npass = -(-ndst // (_NC * _C))
    ndst_pad = npass * _NC * _C
    per_tile = _EP // _NS
    nseg = per_tile // _SEG
    nbat = _SEG // 128
    share = _C // _NS

    def body(vals_hbm, aux_hbm, src_hbm, dst_hbm, sum_out, cnt_out,
             srcseg_v, dstseg_v, idxA_v, idxB_v, rowsA_v, rowsB_v,
             ones_v, acc_sum, acc_cnt, semA, semB):
        c = lax.axis_index("c")
        s = lax.axis_index("s")

        pltpu.sync_copy(aux_hbm.at[pl.ds(128, 128)], ones_v)

        def do_pass(p, carry):
            qbase = (p * _NC + c) * _C
            for k in range(share // 128):
                pltpu.sync_copy(aux_hbm.at[pl.ds(0, 128)],
                                acc_sum.at[pl.ds(s * share + k * 128, 128)])
                pltpu.sync_copy(aux_hbm.at[pl.ds(0, 128)],
                                acc_cnt.at[pl.ds(s * share + k * 128, 128)])
            plsc.subcore_barrier()

            idx_refs = (idxA_v, idxB_v)
            row_refs = (rowsA_v, rowsB_v)
            sems = (semA, semB)

            def stage_idx(b, ref):
                for g in range(8):
                    d = dstseg_v[pl.ds(b * 128 + g * 16, 16)]
                    m = (d >= qbase) & (d < qbase + _C)
                    ref[pl.ds(g * 16, 16)] = jnp.where(m, d - qbase, _C)

            def gather(b, buf):
                return pltpu.async_copy(
                    vals_hbm.at[srcseg_v.at[pl.ds(b * 128, 128)]],
                    row_refs[buf], sems[buf])

            def do_seg(t, carry2):
                pltpu.sync_copy(
                    src_hbm.at[pl.ds(s * per_tile + t * _SEG, _SEG)], srcseg_v)
                pltpu.sync_copy(
                    dst_hbm.at[pl.ds(s * per_tile + t * _SEG, _SEG)], dstseg_v)
                stage_idx(0, idxA_v)
                g0 = gather(0, 0)
                descs = [g0]
                for b in range(nbat):
                    buf = b % 2
                    if b + 1 < nbat:
                        nxt = (b + 1) % 2
                        stage_idx(b + 1, idx_refs[nxt])
                        descs.append(gather(b + 1, nxt))
                    descs[b].wait()
                    pltpu.sync_copy(row_refs[buf],
                                    acc_sum.at[idx_refs[buf]], add=True)
                    pltpu.sync_copy(ones_v,
                                    acc_cnt.at[idx_refs[buf]], add=True)
                return carry2
            lax.fori_loop(0, nseg, do_seg, 0)

            plsc.subcore_barrier()
            dstbase = qbase + s * share
            pltpu.sync_copy(acc_sum.at[pl.ds(s * share, share)],
                            sum_out.at[pl.ds(dstbase, share)])
            pltpu.sync_copy(acc_cnt.at[pl.ds(s * share, share)],
                            cnt_out.at[pl.ds(dstbase, share)])
            plsc.subcore_barrier()
            return carry
        lax.fori_loop(0, npass, do_pass, 0)

    kfn = pl.kernel(
        body,
        out_type=[jax.ShapeDtypeStruct((ndst_pad, _H), jnp.float32),
                  jax.ShapeDtypeStruct((ndst_pad, _H), jnp.float32)],
        mesh=_sc_mesh(),
        scratch_types=[
            pltpu.VMEM((_SEG,), jnp.int32),
            pltpu.VMEM((_SEG,), jnp.int32),
            pltpu.VMEM((128,), jnp.int32),
            pltpu.VMEM((128,), jnp.int32),
            pltpu.VMEM((128, _H), jnp.float32),
            pltpu.VMEM((128, _H), jnp.float32),
            pltpu.VMEM((128, _H), jnp.float32),
            pltpu.VMEM_SHARED((_C + 16, _H), jnp.float32),
            pltpu.VMEM_SHARED((_C + 16, _H), jnp.float32),
            pltpu.SemaphoreType.DMA,
            pltpu.SemaphoreType.DMA,
        ],
    )
    return kfn(vals, aux, src_p, dst_p)


def _edge_gather(u1, u2, src_p, dst_p):
    per_tile = _EP // (_NC * _NS)
    nbatch = per_tile // 128

    def body(u1_hbm, u2_hbm, src_hbm, dst_hbm, e1_out, e2_out,
             sidx_v, didx_v, rows1_v, rows2_v, sem1, sem2):
        c = lax.axis_index("c")
        s = lax.axis_index("s")
        wid = s * _NC + c

        def batch(b, carry):
            off = wid * per_tile + b * 128
            pltpu.sync_copy(src_hbm.at[pl.ds(off, 128)], sidx_v)
            pltpu.sync_copy(dst_hbm.at[pl.ds(off, 128)], didx_v)
            g1 = pltpu.async_copy(u1_hbm.at[sidx_v], rows1_v, sem1)
            g2 = pltpu.async_copy(u2_hbm.at[didx_v], rows2_v, sem2)
            g1.wait()
            pltpu.sync_copy(rows1_v, e1_out.at[pl.ds(off, 128)])
            g2.wait()
            pltpu.sync_copy(rows2_v, e2_out.at[pl.ds(off, 128)])
            return carry
        lax.fori_loop(0, nbatch, batch, 0)

    kfn = pl.kernel(
        body,
        out_type=[jax.ShapeDtypeStruct((_EP, _H), jnp.float32),
                  jax.ShapeDtypeStruct((_EP, _H), jnp.float32)],
        mesh=_sc_mesh(),
        scratch_types=[
            pltpu.VMEM((128,), jnp.int32),
            pltpu.VMEM((128,), jnp.int32),
            pltpu.VMEM((128, _H), jnp.float32),
            pltpu.VMEM((128, _H), jnp.float32),
            pltpu.SemaphoreType.DMA,
            pltpu.SemaphoreType.DMA,
        ],
    )
    return kfn(u1, u2, src_p, dst_p)


def _enc(x8, wx, b):
    n = x8.shape[0]
    B = 1000

    def body(x_ref, w_ref, b_ref, h_ref, c_ref):
        a = jnp.dot(x_ref[...], w_ref[...],
                    preferred_element_type=jnp.float32) + b_ref[...]
        i, f, g, o = (a[:, :_H], a[:, _H:2 * _H],
                      a[:, 2 * _H:3 * _H], a[:, 3 * _H:])
        c1 = jax.nn.sigmoid(i) * jnp.tanh(g)
        h_ref[...] = jax.nn.sigmoid(o) * jnp.tanh(c1)
        c_ref[...] = c1

    return pl.pallas_call(
        body,
        grid=(n // B,),
        in_specs=[pl.BlockSpec((B, 8), lambda i: (i, 0)),
                  pl.BlockSpec((8, 4 * _H), lambda i: (0, 0)),
                  pl.BlockSpec((1, 4 * _H), lambda i: (0, 0))],
        out_specs=[pl.BlockSpec((B, _H), lambda i: (i, 0))] * 2,
        out_shape=[jax.ShapeDtypeStruct((n, _H), jnp.float32)] * 2,
    )(x8, wx, b)


def _dec_joint(x8, h1, c1, m1s, m1c, m2s, m2c,
               wx, wl1, wl2, wr, b, linj8, bj8, ed1a, ed1b, ed1bias):
    B = 1000

    def body(x_ref, h_ref, c_ref, m1s_ref, m1c_ref, m2s_ref, m2c_ref,
             wx_ref, wl1_ref, wl2_ref, wr_ref, b_ref, linj_ref, bj_ref,
             e1a_ref, e1b_ref, e1bias_ref, u1_ref, u2_ref, yj_ref):
        m1 = m1s_ref[...] / jnp.maximum(m1c_ref[:, 0:1], 1.0)
        m2 = m2s_ref[...] / jnp.maximum(m2c_ref[:, 0:1], 1.0)
        dot = functools.partial(jnp.dot, preferred_element_type=jnp.float32)
        a = (dot(x_ref[...], wx_ref[...]) + dot(m1, wl1_ref[...])
             + dot(m2, wl2_ref[...]) + dot(h_ref[...], wr_ref[...])
             + b_ref[...])
        i, f, g, o = (a[:, :_H], a[:, _H:2 * _H],
                      a[:, 2 * _H:3 * _H], a[:, 3 * _H:])
        c2 = jax.nn.sigmoid(f) * c_ref[...] + jax.nn.sigmoid(i) * jnp.tanh(g)
        h2 = jax.nn.sigmoid(o) * jnp.tanh(c2)
        u1_ref[...] = dot(h2, e1a_ref[...]) + e1bias_ref[...]
        u2_ref[...] = dot(h2, e1b_ref[...])
        yj_ref[...] = jax.nn.sigmoid(dot(h2, linj_ref[...]) + bj_ref[...]) - 0.5

    full = lambda shape: pl.BlockSpec(shape, lambda i: (0, 0))
    row = lambda w: pl.BlockSpec((B, w), lambda i: (i, 0))
    return pl.pallas_call(
        body,
        grid=(_NJ // B,),
        in_specs=[row(8), row(_H), row(_H), row(_H), row(_H), row(_H), row(_H),
                  full((8, 4 * _H)), full((_H, 4 * _H)), full((_H, 4 * _H)),
                  full((_H, 4 * _H)), full((1, 4 * _H)), full((_H, 8)),
                  full((1, 8)), full((_H, _H)), full((_H, _H)), full((1, _H))],
        out_specs=[row(_H), row(_H), row(8)],
        out_shape=[jax.ShapeDtypeStruct((_NJ, _H), jnp.float32),
                   jax.ShapeDtypeStruct((_NJ, _H), jnp.float32),
                   jax.ShapeDtypeStruct((_NJ, 8), jnp.float32)],
    )(x8, h1, c1, m1s, m1c, m2s, m2c, wx, wl1, wl2, wr, b, linj8, bj8,
      ed1a, ed1b, ed1bias)


def _dec_grain(x8, h1, c1, ms, mc, wx, wl, wr, b, ling8, bg8):
    B = 1000

    def body(x_ref, h_ref, c_ref, ms_ref, mc_ref,
             wx_ref, wl_ref, wr_ref, b_ref, ling_ref, bg_ref,
             g8_ref, tot_ref):
        m = ms_ref[...] / jnp.maximum(mc_ref[:, 0:1], 1.0)
        dot = functools.partial(jnp.dot, preferred_element_type=jnp.float32)
        a = (dot(x_ref[...], wx_ref[...]) + dot(m, wl_ref[...])
             + dot(h_ref[...], wr_ref[...]) + b_ref[...])
        i, f, g, o = (a[:, :_H], a[:, _H:2 * _H],
                      a[:, 2 * _H:3 * _H], a[:, 3 * _H:])
        c2 = jax.nn.sigmoid(f) * c_ref[...] + jax.nn.sigmoid(i) * jnp.tanh(g)
        h2 = jax.nn.sigmoid(o) * jnp.tanh(c2)
        y8 = dot(h2, ling_ref[...]) + bg_ref[...]
        col = lax.broadcasted_iota(jnp.int32, (B, 8), 1)
        x3 = x_ref[:, 3:4]
        relu_shift = jnp.maximum(y8 + x3, 0.0)
        relu_y = jnp.maximum(y8, 0.0)
        g8_ref[...] = jnp.where(col == 0, relu_shift,
                                jnp.where(col == 1, relu_y, 0.0))
        part = jnp.sum(jnp.where(col == 0, relu_shift, 0.0))

        @pl.when(pl.program_id(0) == 0)
        def _():
            tot_ref[0, 0] = 0.0
        tot_ref[0, 0] += part

    full = lambda shape: pl.BlockSpec(shape, lambda i: (0, 0))
    row = lambda w: pl.BlockSpec((B, w), lambda i: (i, 0))
    return pl.pallas_call(
        body,
        grid=(_NG // B,),
        in_specs=[row(8), row(_H), row(_H), row(_H), row(_H),
                  full((8, 4 * _H)), full((_H, 4 * _H)), full((_H, 4 * _H)),
                  full((1, 4 * _H)), full((_H, 8)), full((1, 8))],
        out_specs=[row(8),
                   pl.BlockSpec((1, 1), lambda i: (0, 0),
                                memory_space=pltpu.SMEM)],
        out_shape=[jax.ShapeDtypeStruct((_NG, 8), jnp.float32),
                   jax.ShapeDtypeStruct((1, 1), jnp.float32)],
    )(x8, h1, c1, ms, mc, wx, wl, wr, b, ling8, bg8)


def _area(g8, x8, tot):
    B = 1000

    def body(g8_ref, x_ref, tot_ref, out_ref):
        t = jnp.maximum(tot_ref[0, 0], 1e-12)
        col = lax.broadcasted_iota(jnp.int32, (B, 8), 1)
        x3 = x_ref[:, 3:4]
        out_ref[...] = jnp.where(col == 0, g8_ref[...] / t - x3, g8_ref[...])

    return pl.pallas_call(
        body,
        grid=(_NG // B,),
        in_specs=[pl.BlockSpec((B, 8), lambda i: (i, 0)),
                  pl.BlockSpec((B, 8), lambda i: (i, 0)),
                  pl.BlockSpec((1, 1), lambda i: (0, 0),
                               memory_space=pltpu.SMEM)],
        out_specs=pl.BlockSpec((B, 8), lambda i: (i, 0)),
        out_shape=jax.ShapeDtypeStruct((_NG, 8), jnp.float32),
    )(g8, x8, tot)


def _edge_final(e1, e2, w2pad, b2pad):
    B = 8192

    def body(e1_ref, e2_ref, w2_ref, b2_ref, z_ref):
        t = jnp.maximum(e1_ref[...] + e2_ref[...], 0.0)
        z_ref[...] = jax.nn.sigmoid(
            jnp.dot(t, w2_ref[...], preferred_element_type=jnp.float32)
            + b2_ref[...])

    return pl.pallas_call(
        body,
        grid=(_EP // B,),
        in_specs=[pl.BlockSpec((B, _H), lambda i: (i, 0)),
                  pl.BlockSpec((B, _H), lambda i: (i, 0)),
                  pl.BlockSpec((_H, 8), lambda i: (0, 0)),
                  pl.BlockSpec((1, 8), lambda i: (0, 0))],
        out_specs=pl.BlockSpec((B, 8), lambda i: (i, 0)),
        out_shape=jax.ShapeDtypeStruct((_EP, 8), jnp.float32),
    )(e1, e2, w2pad, b2pad)


def kernel(x_grain, x_joint, edge_index_gj, edge_index_jg, edge_index_jj,
           params):
    f32 = jnp.float32
    xg8 = x_grain
    xj8 = jnp.pad(x_joint, ((0, 0), (0, 2)))

    def cat_w(cell, kind, pad_rows=0):
        w = jnp.concatenate([params[f'{cell}_{g}_{kind}'] for g in 'ifco'],
                            axis=1)
        if pad_rows:
            w = jnp.pad(w, ((0, pad_rows), (0, 0)))
        return w

    def cat_b(cell, kind):
        return jnp.concatenate(
            [params[f'{cell}_{g}_{kind}'] for g in 'ifco'])[None, :]

    h_g1, c_g1 = _enc(xg8, cat_w('enc', 'Wx_g'), cat_b('enc', 'b_g'))
    h_j1, c_j1 = _enc(xj8, cat_w('enc', 'Wx_j', pad_rows=2),
                      cat_b('enc', 'b_j'))

    aux = jnp.concatenate(
        [jnp.zeros((128, _H), f32),
         jnp.zeros((128, _H), f32).at[:, 0].set(1.0)])

    npad = _EP - _E
    pad0 = jnp.zeros((npad,), jnp.int32)
    sent = jnp.full((npad,), _SENT, jnp.int32)
    gj_src = jnp.concatenate([edge_index_gj[0], pad0])
    gj_dst = jnp.concatenate([edge_index_gj[1], sent])
    jg_src = jnp.concatenate([edge_index_jg[0], pad0])
    jg_dst = jnp.concatenate([edge_index_jg[1], sent])
    jj_src = jnp.concatenate([edge_index_jj[0], pad0])
    jj_dst = jnp.concatenate([edge_index_jj[1], sent])
    jj_src0 = jnp.concatenate([edge_index_jj[0], pad0])
    jj_dst0 = jnp.concatenate([edge_index_jj[1], pad0])

    sj1, nj1 = _segsum(h_g1, aux, gj_src, gj_dst, 50000)
    sj2, nj2 = _segsum(h_j1, aux, jj_src, jj_dst, _NJ)
    sg, ng = _segsum(h_j1, aux, jg_src, jg_dst, _NG)

    zf = jnp.zeros((_NJ - sj1.shape[0], _H), f32)
    m1s = jnp.concatenate([sj1, zf])
    m1c = jnp.concatenate([nj1, zf])
    m2s, m2c = sj2[:_NJ], nj2[:_NJ]
    mgs, mgc = sg[:_NG], ng[:_NG]

    wx_j = cat_w('dec', 'Wx_j', pad_rows=2)
    wl_gj = cat_w('dec', 'Wl_gj')
    wl_jj = cat_w('dec', 'Wl_jj')
    wr_j = jnp.concatenate(
        [params[f'dec_{g}_Wr_gj'] + params[f'dec_{g}_Wr_jj'] for g in 'ifco'],
        axis=1)
    b_j = cat_b('dec', 'b_j')
    wx_g = cat_w('dec', 'Wx_g')
    wl_jg = cat_w('dec', 'Wl_jg')
    wr_g = cat_w('dec', 'Wr_jg')
    b_g = cat_b('dec', 'b_g')

    linj8 = jnp.pad(params['lin_j_W'], ((0, 0), (0, 5)))
    bj8 = jnp.pad(params['lin_j_b'], (0, 5))[None, :]
    ed1a = params['ed1_W'][:_H]
    ed1b = params['ed1_W'][_H:]
    ed1bias = params['ed1_b'][None, :]
    ling8 = jnp.pad(params['lin_g_W'], ((0, 0), (0, 6)))
    bg8 = jnp.pad(params['lin_g_b'], (0, 6))[None, :]
    w2pad = jnp.pad(params['ed2_W'], ((0, 0), (0, 7)))
    b2pad = jnp.pad(params['ed2_b'], (0, 7))[None, :]

    u1, u2, yj8 = _dec_joint(xj8, h_j1, c_j1, m1s, m1c, m2s, m2c,
                             wx_j, wl_gj, wl_jj, wr_j, b_j,
                             linj8, bj8, ed1a, ed1b, ed1bias)
    g8, tot = _dec_grain(xg8, h_g1, c_g1, mgs, mgc,
                         wx_g, wl_jg, wr_g, b_g, ling8, bg8)
    ya8 = _area(g8, xg8, tot)

    e1, e2 = _edge_gather(u1, u2, jj_src0, jj_dst0)
    z8 = _edge_final(e1, e2, w2pad, b2pad)

    return ya8[:, :2], yj8[:, :3], z8[:_E, 0]


# speedup vs baseline: 1.3426x; 1.3108x over previous
import functools

import jax
import jax.numpy as jnp
from jax import lax
from jax.experimental import pallas as pl
from jax.experimental.pallas import tpu as pltpu
from jax.experimental.pallas import tpu_sc as plsc

_NG, _NJ, _H = 50000, 100000, 128
_E, _EP = 200000, 204800
_C = 5120
_NC, _NS = 2, 16
_SENT = 1 << 30
_CAP = 6144
_SEG = 1280


def _sc_mesh():
    return plsc.VectorSubcoreMesh(core_axis_name="c", subcore_axis_name="s",
                                  num_cores=_NC, num_subcores=_NS)


def _segsum(vals, aux, src_p, dst_p, ndst):
    npass = -(-ndst // (_NC * _C))
    ndst_pad = npass * _NC * _C
    per_tile = _EP // _NS
    nseg = per_tile // _SEG
    nbat = _SEG // 128
    share = _C // _NS

    def body(vals_hbm, aux_hbm, src_hbm, dst_hbm, sum_out, cnt_out,
             srcseg_v, dstseg_v, idxA_v, rowsA_v,
             ones_v, acc_sum, acc_cnt, semA):
        c = lax.axis_index("c")
        s = lax.axis_index("s")

        pltpu.sync_copy(aux_hbm.at[pl.ds(128, 128)], ones_v)

        def do_pass(p, carry):
            qbase = (p * _NC + c) * _C
            zoff = 0
            for zblk in ([128] * (share // 128) + ([share % 128] if share % 128 else [])):
                pltpu.sync_copy(aux_hbm.at[pl.ds(0, zblk)],
                                acc_sum.at[pl.ds(s * share + zoff, zblk)])
                pltpu.sync_copy(aux_hbm.at[pl.ds(0, zblk)],
                                acc_cnt.at[pl.ds(s * share + zoff, zblk)])
                zoff += zblk
            plsc.subcore_barrier()

            def stage_idx(b, ref):
                for g in range(8):
                    d = dstseg_v[pl.ds(b * 128 + g * 16, 16)]
                    m = (d >= qbase) & (d < qbase + _C)
                    ref[pl.ds(g * 16, 16)] = jnp.where(m, d - qbase, _C)

            def do_seg(t, carry2):
                pltpu.sync_copy(
                    src_hbm.at[pl.ds(s * per_tile + t * _SEG, _SEG)], srcseg_v)
                pltpu.sync_copy(
                    dst_hbm.at[pl.ds(s * per_tile + t * _SEG, _SEG)], dstseg_v)
                for b in range(nbat):
                    g = pltpu.async_copy(
                        vals_hbm.at[srcseg_v.at[pl.ds(b * 128, 128)]],
                        rowsA_v, semA)
                    stage_idx(b, idxA_v)
                    g.wait()
                    pltpu.sync_copy(rowsA_v, acc_sum.at[idxA_v], add=True)
                    pltpu.sync_copy(ones_v, acc_cnt.at[idxA_v], add=True)
                return carry2
            lax.fori_loop(0, nseg, do_seg, 0)

            plsc.subcore_barrier()
            dstbase = qbase + s * share
            pltpu.sync_copy(acc_sum.at[pl.ds(s * share, share)],
                            sum_out.at[pl.ds(dstbase, share)])
            pltpu.sync_copy(acc_cnt.at[pl.ds(s * share, share)],
                            cnt_out.at[pl.ds(dstbase, share)])
            plsc.subcore_barrier()
            return carry
        lax.fori_loop(0, npass, do_pass, 0)

    kfn = pl.kernel(
        body,
        out_type=[jax.ShapeDtypeStruct((ndst_pad, _H), jnp.float32),
                  jax.ShapeDtypeStruct((ndst_pad, _H), jnp.float32)],
        mesh=_sc_mesh(),
        scratch_types=[
            pltpu.VMEM((_SEG,), jnp.int32),
            pltpu.VMEM((_SEG,), jnp.int32),
            pltpu.VMEM((128,), jnp.int32),
            pltpu.VMEM((128, _H), jnp.float32),
            pltpu.VMEM((128, _H), jnp.float32),
            pltpu.VMEM_SHARED((_C + 16, _H), jnp.float32),
            pltpu.VMEM_SHARED((_C + 16, _H), jnp.float32),
            pltpu.SemaphoreType.DMA,
        ],
    )
    return kfn(vals, aux, src_p, dst_p)


def _edge_gather(u1, u2, src_p, dst_p):
    per_tile = _EP // (_NC * _NS)
    nbatch = per_tile // 128

    def body(u1_hbm, u2_hbm, src_hbm, dst_hbm, e1_out, e2_out,
             sidx_v, didx_v, rows1_v, rows2_v, sem1, sem2):
        c = lax.axis_index("c")
        s = lax.axis_index("s")
        wid = s * _NC + c

        def batch(b, carry):
            off = wid * per_tile + b * 128
            pltpu.sync_copy(src_hbm.at[pl.ds(off, 128)], sidx_v)
            pltpu.sync_copy(dst_hbm.at[pl.ds(off, 128)], didx_v)
            g1 = pltpu.async_copy(u1_hbm.at[sidx_v], rows1_v, sem1)
            g2 = pltpu.async_copy(u2_hbm.at[didx_v], rows2_v, sem2)
            g1.wait()
            pltpu.sync_copy(rows1_v, e1_out.at[pl.ds(off, 128)])
            g2.wait()
            pltpu.sync_copy(rows2_v, e2_out.at[pl.ds(off, 128)])
            return carry
        lax.fori_loop(0, nbatch, batch, 0)

    kfn = pl.kernel(
        body,
        out_type=[jax.ShapeDtypeStruct((_EP, _H), jnp.float32),
                  jax.ShapeDtypeStruct((_EP, _H), jnp.float32)],
        mesh=_sc_mesh(),
        scratch_types=[
            pltpu.VMEM((128,), jnp.int32),
            pltpu.VMEM((128,), jnp.int32),
            pltpu.VMEM((128, _H), jnp.float32),
            pltpu.VMEM((128, _H), jnp.float32),
            pltpu.SemaphoreType.DMA,
            pltpu.SemaphoreType.DMA,
        ],
    )
    return kfn(u1, u2, src_p, dst_p)


def _enc(x8, wx, b):
    n = x8.shape[0]
    B = 1000

    def body(x_ref, w_ref, b_ref, h_ref, c_ref):
        a = jnp.dot(x_ref[...], w_ref[...],
                    preferred_element_type=jnp.float32) + b_ref[...]
        i, f, g, o = (a[:, :_H], a[:, _H:2 * _H],
                      a[:, 2 * _H:3 * _H], a[:, 3 * _H:])
        c1 = jax.nn.sigmoid(i) * jnp.tanh(g)
        h_ref[...] = jax.nn.sigmoid(o) * jnp.tanh(c1)
        c_ref[...] = c1

    return pl.pallas_call(
        body,
        grid=(n // B,),
        in_specs=[pl.BlockSpec((B, 8), lambda i: (i, 0)),
                  pl.BlockSpec((8, 4 * _H), lambda i: (0, 0)),
                  pl.BlockSpec((1, 4 * _H), lambda i: (0, 0))],
        out_specs=[pl.BlockSpec((B, _H), lambda i: (i, 0))] * 2,
        out_shape=[jax.ShapeDtypeStruct((n, _H), jnp.float32)] * 2,
    )(x8, wx, b)


def _dec_joint(x8, h1, c1, m1s, m1c, m2s, m2c,
               wx, wl1, wl2, wr, b, linj8, bj8, ed1a, ed1b, ed1bias):
    B = 1000

    def body(x_ref, h_ref, c_ref, m1s_ref, m1c_ref, m2s_ref, m2c_ref,
             wx_ref, wl1_ref, wl2_ref, wr_ref, b_ref, linj_ref, bj_ref,
             e1a_ref, e1b_ref, e1bias_ref, u1_ref, u2_ref, yj_ref):
        m1 = m1s_ref[...] / jnp.maximum(m1c_ref[:, 0:1], 1.0)
        m2 = m2s_ref[...] / jnp.maximum(m2c_ref[:, 0:1], 1.0)
        dot = functools.partial(jnp.dot, preferred_element_type=jnp.float32)
        a = (dot(x_ref[...], wx_ref[...]) + dot(m1, wl1_ref[...])
             + dot(m2, wl2_ref[...]) + dot(h_ref[...], wr_ref[...])
             + b_ref[...])
        i, f, g, o = (a[:, :_H], a[:, _H:2 * _H],
                      a[:, 2 * _H:3 * _H], a[:, 3 * _H:])
        c2 = jax.nn.sigmoid(f) * c_ref[...] + jax.nn.sigmoid(i) * jnp.tanh(g)
        h2 = jax.nn.sigmoid(o) * jnp.tanh(c2)
        u1_ref[...] = dot(h2, e1a_ref[...]) + e1bias_ref[...]
        u2_ref[...] = dot(h2, e1b_ref[...])
        yj_ref[...] = jax.nn.sigmoid(dot(h2, linj_ref[...]) + bj_ref[...]) - 0.5

    full = lambda shape: pl.BlockSpec(shape, lambda i: (0, 0))
    row = lambda w: pl.BlockSpec((B, w), lambda i: (i, 0))
    return pl.pallas_call(
        body,
        grid=(_NJ // B,),
        in_specs=[row(8), row(_H), row(_H), row(_H), row(_H), row(_H), row(_H),
                  full((8, 4 * _H)), full((_H, 4 * _H)), full((_H, 4 * _H)),
                  full((_H, 4 * _H)), full((1, 4 * _H)), full((_H, 8)),
                  full((1, 8)), full((_H, _H)), full((_H, _H)), full((1, _H))],
        out_specs=[row(_H), row(_H), row(8)],
        out_shape=[jax.ShapeDtypeStruct((_NJ, _H), jnp.float32),
                   jax.ShapeDtypeStruct((_NJ, _H), jnp.float32),
                   jax.ShapeDtypeStruct((_NJ, 8), jnp.float32)],
    )(x8, h1, c1, m1s, m1c, m2s, m2c, wx, wl1, wl2, wr, b, linj8, bj8,
      ed1a, ed1b, ed1bias)


def _dec_grain(x8, h1, c1, ms, mc, wx, wl, wr, b, ling8, bg8):
    B = 1000

    def body(x_ref, h_ref, c_ref, ms_ref, mc_ref,
             wx_ref, wl_ref, wr_ref, b_ref, ling_ref, bg_ref,
             g8_ref, tot_ref):
        m = ms_ref[...] / jnp.maximum(mc_ref[:, 0:1], 1.0)
        dot = functools.partial(jnp.dot, preferred_element_type=jnp.float32)
        a = (dot(x_ref[...], wx_ref[...]) + dot(m, wl_ref[...])
             + dot(h_ref[...], wr_ref[...]) + b_ref[...])
        i, f, g, o = (a[:, :_H], a[:, _H:2 * _H],
                      a[:, 2 * _H:3 * _H], a[:, 3 * _H:])
        c2 = jax.nn.sigmoid(f) * c_ref[...] + jax.nn.sigmoid(i) * jnp.tanh(g)
        h2 = jax.nn.sigmoid(o) * jnp.tanh(c2)
        y8 = dot(h2, ling_ref[...]) + bg_ref[...]
        col = lax.broadcasted_iota(jnp.int32, (B, 8), 1)
        x3 = x_ref[:, 3:4]
        relu_shift = jnp.maximum(y8 + x3, 0.0)
        relu_y = jnp.maximum(y8, 0.0)
        g8_ref[...] = jnp.where(col == 0, relu_shift,
                                jnp.where(col == 1, relu_y, 0.0))
        part = jnp.sum(jnp.where(col == 0, relu_shift, 0.0))

        @pl.when(pl.program_id(0) == 0)
        def _():
            tot_ref[0, 0] = 0.0
        tot_ref[0, 0] += part

    full = lambda shape: pl.BlockSpec(shape, lambda i: (0, 0))
    row = lambda w: pl.BlockSpec((B, w), lambda i: (i, 0))
    return pl.pallas_call(
        body,
        grid=(_NG // B,),
        in_specs=[row(8), row(_H), row(_H), row(_H), row(_H),
                  full((8, 4 * _H)), full((_H, 4 * _H)), full((_H, 4 * _H)),
                  full((1, 4 * _H)), full((_H, 8)), full((1, 8))],
        out_specs=[row(8),
                   pl.BlockSpec((1, 1), lambda i: (0, 0),
                                memory_space=pltpu.SMEM)],
        out_shape=[jax.ShapeDtypeStruct((_NG, 8), jnp.float32),
                   jax.ShapeDtypeStruct((1, 1), jnp.float32)],
    )(x8, h1, c1, ms, mc, wx, wl, wr, b, ling8, bg8)


def _area(g8, x8, tot):
    B = 1000

    def body(g8_ref, x_ref, tot_ref, out_ref):
        t = jnp.maximum(tot_ref[0, 0], 1e-12)
        col = lax.broadcasted_iota(jnp.int32, (B, 8), 1)
        x3 = x_ref[:, 3:4]
        out_ref[...] = jnp.where(col == 0, g8_ref[...] / t - x3, g8_ref[...])

    return pl.pallas_call(
        body,
        grid=(_NG // B,),
        in_specs=[pl.BlockSpec((B, 8), lambda i: (i, 0)),
                  pl.BlockSpec((B, 8), lambda i: (i, 0)),
                  pl.BlockSpec((1, 1), lambda i: (0, 0),
                               memory_space=pltpu.SMEM)],
        out_specs=pl.BlockSpec((B, 8), lambda i: (i, 0)),
        out_shape=jax.ShapeDtypeStruct((_NG, 8), jnp.float32),
    )(g8, x8, tot)


def _edge_final(e1, e2, w2pad, b2pad):
    B = 8192

    def body(e1_ref, e2_ref, w2_ref, b2_ref, z_ref):
        t = jnp.maximum(e1_ref[...] + e2_ref[...], 0.0)
        z_ref[...] = jax.nn.sigmoid(
            jnp.dot(t, w2_ref[...], preferred_element_type=jnp.float32)
            + b2_ref[...])

    return pl.pallas_call(
        body,
        grid=(_EP // B,),
        in_specs=[pl.BlockSpec((B, _H), lambda i: (i, 0)),
                  pl.BlockSpec((B, _H), lambda i: (i, 0)),
                  pl.BlockSpec((_H, 8), lambda i: (0, 0)),
                  pl.BlockSpec((1, 8), lambda i: (0, 0))],
        out_specs=pl.BlockSpec((B, 8), lambda i: (i, 0)),
        out_shape=jax.ShapeDtypeStruct((_EP, 8), jnp.float32),
    )(e1, e2, w2pad, b2pad)


def kernel(x_grain, x_joint, edge_index_gj, edge_index_jg, edge_index_jj,
           params):
    f32 = jnp.float32
    xg8 = x_grain
    xj8 = jnp.pad(x_joint, ((0, 0), (0, 2)))

    def cat_w(cell, kind, pad_rows=0):
        w = jnp.concatenate([params[f'{cell}_{g}_{kind}'] for g in 'ifco'],
                            axis=1)
        if pad_rows:
            w = jnp.pad(w, ((0, pad_rows), (0, 0)))
        return w

    def cat_b(cell, kind):
        return jnp.concatenate(
            [params[f'{cell}_{g}_{kind}'] for g in 'ifco'])[None, :]

    h_g1, c_g1 = _enc(xg8, cat_w('enc', 'Wx_g'), cat_b('enc', 'b_g'))
    h_j1, c_j1 = _enc(xj8, cat_w('enc', 'Wx_j', pad_rows=2),
                      cat_b('enc', 'b_j'))

    aux = jnp.concatenate(
        [jnp.zeros((128, _H), f32),
         jnp.zeros((128, _H), f32).at[:, 0].set(1.0)])

    npad = _EP - _E
    pad0 = jnp.zeros((npad,), jnp.int32)
    sent = jnp.full((npad,), _SENT, jnp.int32)
    gj_src = jnp.concatenate([edge_index_gj[0], pad0])
    gj_dst = jnp.concatenate([edge_index_gj[1], sent])
    jg_src = jnp.concatenate([edge_index_jg[0], pad0])
    jg_dst = jnp.concatenate([edge_index_jg[1], sent])
    jj_src = jnp.concatenate([edge_index_jj[0], pad0])
    jj_dst = jnp.concatenate([edge_index_jj[1], sent])
    jj_src0 = jnp.concatenate([edge_index_jj[0], pad0])
    jj_dst0 = jnp.concatenate([edge_index_jj[1], pad0])

    sj1, nj1 = _segsum(h_g1, aux, gj_src, gj_dst, 50000)
    sj2, nj2 = _segsum(h_j1, aux, jj_src, jj_dst, _NJ)
    sg, ng = _segsum(h_j1, aux, jg_src, jg_dst, _NG)

    zf = jnp.zeros((_NJ - sj1.shape[0], _H), f32)
    m1s = jnp.concatenate([sj1, zf])
    m1c = jnp.concatenate([nj1, zf])
    m2s, m2c = sj2[:_NJ], nj2[:_NJ]
    mgs, mgc = sg[:_NG], ng[:_NG]

    wx_j = cat_w('dec', 'Wx_j', pad_rows=2)
    wl_gj = cat_w('dec', 'Wl_gj')
    wl_jj = cat_w('dec', 'Wl_jj')
    wr_j = jnp.concatenate(
        [params[f'dec_{g}_Wr_gj'] + params[f'dec_{g}_Wr_jj'] for g in 'ifco'],
        axis=1)
    b_j = cat_b('dec', 'b_j')
    wx_g = cat_w('dec', 'Wx_g')
    wl_jg = cat_w('dec', 'Wl_jg')
    wr_g = cat_w('dec', 'Wr_jg')
    b_g = cat_b('dec', 'b_g')

    linj8 = jnp.pad(params['lin_j_W'], ((0, 0), (0, 5)))
    bj8 = jnp.pad(params['lin_j_b'], (0, 5))[None, :]
    ed1a = params['ed1_W'][:_H]
    ed1b = params['ed1_W'][_H:]
    ed1bias = params['ed1_b'][None, :]
    ling8 = jnp.pad(params['lin_g_W'], ((0, 0), (0, 6)))
    bg8 = jnp.pad(params['lin_g_b'], (0, 6))[None, :]
    w2pad = jnp.pad(params['ed2_W'], ((0, 0), (0, 7)))
    b2pad = jnp.pad(params['ed2_b'], (0, 7))[None, :]

    u1, u2, yj8 = _dec_joint(xj8, h_j1, c_j1, m1s, m1c, m2s, m2c,
                             wx_j, wl_gj, wl_jj, wr_j, b_j,
                             linj8, bj8, ed1a, ed1b, ed1bias)
    g8, tot = _dec_grain(xg8, h_g1, c_g1, mgs, mgc,
                         wx_g, wl_jg, wr_g, b_g, ling8, bg8)
    ya8 = _area(g8, xg8, tot)

    e1, e2 = _edge_gather(u1, u2, jj_src0, jj_dst0)
    z8 = _edge_final(e1, e2, w2pad, b2pad)

    return ya8[:, :2], yj8[:, :3], z8[:_E, 0]
